# baseline (speedup 1.0000x reference)
import functools

import jax
import jax.numpy as jnp
from jax import lax
from jax.experimental import pallas as pl
from jax.experimental.pallas import tpu as pltpu
from jax.experimental.pallas import tpu_sc as plsc

NC = 2
NS = 16
CHUNK = 128
BLK = 256
L = 16


def _mesh():
    return plsc.VectorSubcoreMesh(
        core_axis_name="c", subcore_axis_name="s", num_cores=NC, num_subcores=NS
    )


def _rsqrt_newton(x):
    i = plsc.bitcast(x, jnp.int32)
    i = jnp.int32(0x5F3759DF) - lax.shift_right_logical(i, 1)
    y = plsc.bitcast(i, jnp.float32)
    for _ in range(3):
        y = y * (1.5 - 0.5 * x * y * y)
    return y


def _gcn_kernel(R, C, H):
    rows_per = R // NS
    n_pieces = rows_per // CHUNK

    @functools.partial(
        pl.kernel,
        out_type=(
            jax.ShapeDtypeStruct((NC, R, H), jnp.float32),
            jax.ShapeDtypeStruct((NC, R), jnp.float32),
        ),
        mesh=_mesh(),
        scratch_types=[
            pltpu.VMEM(((C + 1) // 2, CHUNK), jnp.int32),
            pltpu.VMEM((C, CHUNK), jnp.int32),
            pltpu.VMEM((CHUNK, H), jnp.float32),
            pltpu.VMEM((rows_per,), jnp.float32),
            pltpu.VMEM((CHUNK,), jnp.float32),
            pltpu.SemaphoreType.DMA,
            pltpu.VMEM_SHARED((R, H), jnp.float32),
            pltpu.VMEM_SHARED((R, H), jnp.float32),
            pltpu.VMEM_SHARED((R,), jnp.float32),
        ],
        compiler_params=pltpu.CompilerParams(use_tc_tiling_on_sc=False,
                                             needs_layout_passes=False),
    )
    def gcn(src_hbm, dst_hbm, feat_hbm, zrow_hbm, z2d_hbm, ones_hbm,
            out_hbm, deg_hbm,
            sidx_v, didx_v, gbuf, cvec_v, ones_v, gsem,
            acc_sh, feat_sh, deg_sh):
        c = lax.axis_index("c")
        s = lax.axis_index("s")
        base = s * rows_per

        pltpu.sync_copy(dst_hbm.at[s], didx_v)
        pltpu.sync_copy(ones_hbm, ones_v)
        pltpu.sync_copy(zrow_hbm.at[pl.ds(base, rows_per)],
                        deg_sh.at[pl.ds(base, rows_per)])
        pltpu.sync_copy(z2d_hbm, acc_sh.at[pl.ds(base, rows_per)])
        plsc.subcore_barrier()

        def dbody(g, carry):
            pltpu.sync_copy(ones_v, deg_sh.at[didx_v.at[g]], add=True)
            return carry

        lax.fori_loop(0, C, dbody, 0)
        plsc.subcore_barrier()
        pltpu.sync_copy(deg_sh.at[pl.ds(base, rows_per)],
                        deg_hbm.at[c, pl.ds(base, rows_per)])
        pltpu.sync_copy(deg_sh.at[pl.ds(base, rows_per)], cvec_v)

        for v in range(rows_per // L):
            cvec_v[pl.ds(v * L, L)] = _rsqrt_newton(
                cvec_v[pl.ds(v * L, L)] + 1.0)

        for p in range(n_pieces):
            off = p * CHUNK
            pltpu.sync_copy(feat_hbm.at[c, pl.ds(base + off, CHUNK)], gbuf)

            def srow(gi, carry):
                cwin = cvec_v[pl.ds(off + gi * L, L)]
                for lane in range(L):
                    bc = lax.gather(
                        cwin, jnp.full((L, 1), lane, jnp.int32),
                        lax.GatherDimensionNumbers(
                            offset_dims=(), collapsed_slice_dims=(0,),
                            start_index_map=(0,)),
                        (1,), mode=lax.GatherScatterMode.PROMISE_IN_BOUNDS)
                    for k in range(H // L):
                        gbuf[gi * L + lane, pl.ds(k * L, L)] = (
                            gbuf[gi * L + lane, pl.ds(k * L, L)] * bc)
                return carry

            lax.fori_loop(0, CHUNK // L, srow, 0)
            pltpu.sync_copy(gbuf, feat_sh.at[pl.ds(base + off, CHUNK)])
        plsc.subcore_barrier()

        CH = (C + 1) // 2

        def half(d0, n):
            pltpu.sync_copy(src_hbm.at[s, pl.ds(d0, n)],
                            sidx_v.at[pl.ds(0, n)])

            def body(g, carry):
                pltpu.async_copy(feat_sh.at[sidx_v.at[g]], gbuf, gsem).wait()
                pltpu.sync_copy(gbuf, acc_sh.at[didx_v.at[d0 + g]], add=True)
                return carry

            lax.fori_loop(0, n, body, 0)

        half(0, CH)
        half(CH, C - CH)
        plsc.subcore_barrier()
        pltpu.sync_copy(acc_sh.at[pl.ds(base, rows_per)],
                        out_hbm.at[c, pl.ds(base, rows_per)])

    return gcn


def _out_body(p_ref, dp_ref, w_ref, b_ref, o_ref):
    i = pl.program_id(0)
    deg = dp_ref[0, pl.ds(i * BLK, BLK)]
    dinv = 1.0 / jnp.maximum(deg, 1.0)
    h = jnp.concatenate([p_ref[0], p_ref[1]], axis=1) * dinv[:, None]
    o_ref[...] = lax.dot_general(
        h, w_ref[...], (((1,), (1,)), ((), ())),
        preferred_element_type=jnp.float32,
    ) + b_ref[...]


def kernel(feature, edge_index, W, b):
    N, D = feature.shape
    H = D // 2
    E = edge_index.shape[1]
    R = ((N + 1 + BLK - 1) // BLK) * BLK

    src = edge_index[0]
    dst = edge_index[1]

    per_t = -(-E // NS)
    C = -(-per_t // CHUNK)
    pad1 = NS * C * CHUNK - E
    padv = jnp.full((pad1,), N, jnp.int32)
    srcf = jnp.concatenate([src, padv]).reshape(NS, C, CHUNK)
    dstf = jnp.concatenate([dst, padv]).reshape(NS, C, CHUNK)

    ones = jnp.ones((CHUNK,), jnp.float32)
    zrow = jnp.zeros((R,), jnp.float32)
    z2d = jnp.zeros((R // NS, H), jnp.float32)
    featsp = jnp.pad(feature.reshape(N, NC, H).transpose(1, 0, 2),
                     ((0, 0), (0, R - N), (0, 0)))

    partial, dp = _gcn_kernel(R, C, H)(srcf, dstf, featsp, zrow, z2d, ones)

    outp = pl.pallas_call(
        _out_body,
        grid=(R // BLK,),
        in_specs=[
            pl.BlockSpec((NC, BLK, H), lambda i: (0, i, 0)),
            pl.BlockSpec((NC, R), lambda i: (0, 0)),
            pl.BlockSpec((D, D), lambda i: (0, 0)),
            pl.BlockSpec((1, D), lambda i: (0, 0)),
        ],
        out_specs=pl.BlockSpec((BLK, D), lambda i: (i, 0)),
        out_shape=jax.ShapeDtypeStruct((N, D), jnp.float32),
    )(partial, dp, W, b.reshape(1, D))

    return outp


# speedup vs baseline: 1.0409x; 1.0409x over previous
import functools

import jax
import jax.numpy as jnp
from jax import lax
from jax.experimental import pallas as pl
from jax.experimental.pallas import tpu as pltpu
from jax.experimental.pallas import tpu_sc as plsc

NC = 2
NS = 16
CHUNK = 128
BLK = 256
L = 16


def _mesh():
    return plsc.VectorSubcoreMesh(
        core_axis_name="c", subcore_axis_name="s", num_cores=NC, num_subcores=NS
    )


def _rsqrt_newton(x):
    i = plsc.bitcast(x, jnp.int32)
    i = jnp.int32(0x5F3759DF) - lax.shift_right_logical(i, 1)
    y = plsc.bitcast(i, jnp.float32)
    for _ in range(3):
        y = y * (1.5 - 0.5 * x * y * y)
    return y


def _deg_kernel(R, C):
    NW = NC * NS
    rows_per = R // NS

    @functools.partial(
        pl.kernel,
        out_type=jax.ShapeDtypeStruct((NC, R), jnp.float32),
        mesh=_mesh(),
        scratch_types=[
            pltpu.VMEM((C, CHUNK), jnp.int32),
            pltpu.VMEM((CHUNK,), jnp.float32),
            pltpu.VMEM_SHARED((R,), jnp.float32),
        ],
    )
    def deg(dst_hbm, ones_hbm, zrow_hbm, out_hbm, idx_v, ones_v, deg_sh):
        c = lax.axis_index("c")
        s = lax.axis_index("s")
        wid = c * NS + s
        base = s * rows_per
        pltpu.sync_copy(zrow_hbm.at[pl.ds(base, rows_per)],
                        deg_sh.at[pl.ds(base, rows_per)])
        pltpu.sync_copy(ones_hbm, ones_v)
        pltpu.sync_copy(dst_hbm.at[wid], idx_v)
        plsc.subcore_barrier()

        def body(g, carry):
            pltpu.sync_copy(ones_v, deg_sh.at[idx_v.at[g]], add=True)
            return carry

        lax.fori_loop(0, C, body, 0)
        plsc.subcore_barrier()
        pltpu.sync_copy(deg_sh.at[pl.ds(base, rows_per)],
                        out_hbm.at[c, pl.ds(base, rows_per)])

    return deg


def _gcn_kernel(R, C, H):
    rows_per = R // NS
    n_pieces = rows_per // CHUNK

    @functools.partial(
        pl.kernel,
        out_type=jax.ShapeDtypeStruct((NC, R, H), jnp.float32),
        mesh=_mesh(),
        scratch_types=[
            pltpu.VMEM(((C + 1) // 2, CHUNK), jnp.int32),
            pltpu.VMEM((C, CHUNK), jnp.int32),
            pltpu.VMEM((CHUNK, H), jnp.float32),
            pltpu.VMEM((rows_per,), jnp.float32),
            pltpu.VMEM((rows_per,), jnp.float32),
            pltpu.SemaphoreType.DMA,
            pltpu.VMEM_SHARED((R, H), jnp.float32),
            pltpu.VMEM_SHARED((R, H), jnp.float32),
        ],
        compiler_params=pltpu.CompilerParams(use_tc_tiling_on_sc=False,
                                             needs_layout_passes=False),
    )
    def gcn(src_hbm, dst_hbm, feat_hbm, dp_hbm, z2d_hbm,
            out_hbm,
            sidx_v, didx_v, gbuf, cvec_v, c2_v, gsem,
            acc_sh, feat_sh):
        c = lax.axis_index("c")
        s = lax.axis_index("s")
        base = s * rows_per

        pltpu.sync_copy(dst_hbm.at[s], didx_v)
        pltpu.sync_copy(z2d_hbm, acc_sh.at[pl.ds(base, rows_per)])
        pltpu.sync_copy(dp_hbm.at[0, pl.ds(base, rows_per)], cvec_v)
        pltpu.sync_copy(dp_hbm.at[1, pl.ds(base, rows_per)], c2_v)

        for v in range(rows_per // L):
            cvec_v[pl.ds(v * L, L)] = _rsqrt_newton(
                cvec_v[pl.ds(v * L, L)] + c2_v[pl.ds(v * L, L)] + 1.0)

        for p in range(n_pieces):
            off = p * CHUNK
            pltpu.sync_copy(feat_hbm.at[c, pl.ds(base + off, CHUNK)], gbuf)

            def srow(gi, carry):
                cwin = cvec_v[pl.ds(off + gi * L, L)]
                for lane in range(L):
                    bc = lax.gather(
                        cwin, jnp.full((L, 1), lane, jnp.int32),
                        lax.GatherDimensionNumbers(
                            offset_dims=(), collapsed_slice_dims=(0,),
                            start_index_map=(0,)),
                        (1,), mode=lax.GatherScatterMode.PROMISE_IN_BOUNDS)
                    for k in range(H // L):
                        gbuf[gi * L + lane, pl.ds(k * L, L)] = (
                            gbuf[gi * L + lane, pl.ds(k * L, L)] * bc)
                return carry

            lax.fori_loop(0, CHUNK // L, srow, 0)
            pltpu.sync_copy(gbuf, feat_sh.at[pl.ds(base + off, CHUNK)])
        plsc.subcore_barrier()

        CH = (C + 1) // 2

        def half(d0, n):
            pltpu.sync_copy(src_hbm.at[s, pl.ds(d0, n)],
                            sidx_v.at[pl.ds(0, n)])

            def body(g, carry):
                pltpu.async_copy(feat_sh.at[sidx_v.at[g]], gbuf, gsem).wait()
                pltpu.sync_copy(gbuf, acc_sh.at[didx_v.at[d0 + g]], add=True)
                return carry

            lax.fori_loop(0, n, body, 0)

        half(0, CH)
        half(CH, C - CH)
        plsc.subcore_barrier()
        pltpu.sync_copy(acc_sh.at[pl.ds(base, rows_per)],
                        out_hbm.at[c, pl.ds(base, rows_per)])

    return gcn


def _out_body(p_ref, dp_ref, w_ref, b_ref, o_ref):
    i = pl.program_id(0)
    deg = dp_ref[0, pl.ds(i * BLK, BLK)] + dp_ref[1, pl.ds(i * BLK, BLK)]
    dinv = 1.0 / jnp.maximum(deg, 1.0)
    h = jnp.concatenate([p_ref[0], p_ref[1]], axis=1) * dinv[:, None]
    o_ref[...] = lax.dot_general(
        h, w_ref[...], (((1,), (1,)), ((), ())),
        preferred_element_type=jnp.float32,
    ) + b_ref[...]


def kernel(feature, edge_index, W, b):
    N, D = feature.shape
    H = D // 2
    E = edge_index.shape[1]
    R = ((N + 1 + BLK - 1) // BLK) * BLK

    src = edge_index[0]
    dst = edge_index[1]

    NW = NC * NS
    per_w = -(-E // NW)
    C0 = -(-per_w // CHUNK)
    pad0 = NW * C0 * CHUNK - E
    dstp32 = jnp.concatenate(
        [dst, jnp.full((pad0,), N, jnp.int32)]).reshape(NW, C0, CHUNK)

    per_t = -(-E // NS)
    C = -(-per_t // CHUNK)
    pad1 = NS * C * CHUNK - E
    padv = jnp.full((pad1,), N, jnp.int32)
    srcf = jnp.concatenate([src, padv]).reshape(NS, C, CHUNK)
    dstf = jnp.concatenate([dst, padv]).reshape(NS, C, CHUNK)

    ones = jnp.ones((CHUNK,), jnp.float32)
    zrow = jnp.zeros((R,), jnp.float32)
    z2d = jnp.zeros((R // NS, H), jnp.float32)
    featsp = jnp.pad(feature.reshape(N, NC, H).transpose(1, 0, 2),
                     ((0, 0), (0, R - N), (0, 0)))

    dp = _deg_kernel(R, C0)(dstp32, ones, zrow)
    partial = _gcn_kernel(R, C, H)(srcf, dstf, featsp, dp, z2d)

    outp = pl.pallas_call(
        _out_body,
        grid=(R // BLK,),
        in_specs=[
            pl.BlockSpec((NC, BLK, H), lambda i: (0, i, 0)),
            pl.BlockSpec((NC, R), lambda i: (0, 0)),
            pl.BlockSpec((D, D), lambda i: (0, 0)),
            pl.BlockSpec((1, D), lambda i: (0, 0)),
        ],
        out_specs=pl.BlockSpec((BLK, D), lambda i: (i, 0)),
        out_shape=jax.ShapeDtypeStruct((N, D), jnp.float32),
    )(partial, dp, W, b.reshape(1, D))

    return outp


# speedup vs baseline: 1.0461x; 1.0050x over previous
import functools

import jax
import jax.numpy as jnp
from jax import lax
from jax.experimental import pallas as pl
from jax.experimental.pallas import tpu as pltpu
from jax.experimental.pallas import tpu_sc as plsc

NC = 2
NS = 16
NW = NC * NS
CHUNK = 128
BLK = 256


def _mesh():
    return plsc.VectorSubcoreMesh(
        core_axis_name="c", subcore_axis_name="s", num_cores=NC, num_subcores=NS
    )


def _deg_kernel(R, C):
    rows_per = R // NS

    @functools.partial(
        pl.kernel,
        out_type=jax.ShapeDtypeStruct((NC, R), jnp.float32),
        mesh=_mesh(),
        scratch_types=[
            pltpu.VMEM((C, CHUNK), jnp.int32),
            pltpu.VMEM((CHUNK,), jnp.float32),
            pltpu.VMEM((rows_per,), jnp.float32),
            pltpu.VMEM_SHARED((R,), jnp.float32),
        ],
    )
    def deg(dst_hbm, ones_hbm, zrow_hbm, out_hbm, idx_v, ones_v, stage_v, deg_sh):
        c = lax.axis_index("c")
        s = lax.axis_index("s")
        wid = c * NS + s
        base = s * rows_per
        pltpu.sync_copy(zrow_hbm.at[pl.ds(base, rows_per)],
                        deg_sh.at[pl.ds(base, rows_per)])
        pltpu.sync_copy(ones_hbm, ones_v)
        pltpu.sync_copy(dst_hbm.at[wid], idx_v)
        plsc.subcore_barrier()

        def body(g, carry):
            pltpu.sync_copy(ones_v, deg_sh.at[idx_v.at[g]], add=True)
            return carry

        lax.fori_loop(0, C, body, 0)
        plsc.subcore_barrier()
        pltpu.sync_copy(deg_sh.at[pl.ds(base, rows_per)], stage_v)
        pltpu.sync_copy(stage_v, out_hbm.at[c, pl.ds(base, rows_per)])

    return deg


def _msg_kernel(R, C, H):
    rows_per = R // NS

    @functools.partial(
        pl.kernel,
        out_type=jax.ShapeDtypeStruct((NC, R, H), jnp.float32),
        mesh=_mesh(),
        scratch_types=[
            pltpu.VMEM((C, CHUNK), jnp.int32),
            pltpu.VMEM((C, CHUNK), jnp.int32),
            pltpu.VMEM((CHUNK, H), jnp.float32),
            pltpu.SemaphoreType.DMA,
            pltpu.VMEM_SHARED((R, H), jnp.float32),
            pltpu.VMEM_SHARED((R, H), jnp.float32),
        ],
        compiler_params=pltpu.CompilerParams(use_tc_tiling_on_sc=False),
    )
    def msg(src_hbm, dst_hbm, feat_hbm, zrows_hbm, out_hbm,
            sidx_v, didx_v, gbuf, gsem, acc_sh, feat_sh):
        c = lax.axis_index("c")
        s = lax.axis_index("s")
        base = s * rows_per

        pltpu.sync_copy(src_hbm.at[s], sidx_v)
        pltpu.sync_copy(dst_hbm.at[s], didx_v)
        pltpu.sync_copy(feat_hbm.at[c, pl.ds(base, rows_per)],
                        feat_sh.at[pl.ds(base, rows_per)])
        pltpu.sync_copy(zrows_hbm, acc_sh.at[pl.ds(base, rows_per)])
        plsc.subcore_barrier()

        def body(g, carry):
            pltpu.async_copy(feat_sh.at[sidx_v.at[g]],
                             gbuf, gsem).wait()
            pltpu.sync_copy(gbuf, acc_sh.at[didx_v.at[g]], add=True)
            return carry

        lax.fori_loop(0, C, body, 0)
        plsc.subcore_barrier()
        pltpu.sync_copy(acc_sh.at[pl.ds(base, rows_per)],
                        out_hbm.at[c, pl.ds(base, rows_per)])

    return msg


def _prep_body(dp_ref, feat_ref, out_ref):
    i = pl.program_id(0)
    deg = dp_ref[0, pl.ds(i * BLK, BLK)] + dp_ref[1, pl.ds(i * BLK, BLK)]
    scale = lax.rsqrt(deg + 1.0)
    scaled = feat_ref[...] * scale[:, None]
    h = scaled.shape[1] // 2
    out_ref[0] = scaled[:, :h]
    out_ref[1] = scaled[:, h:]


def _out_body(p_ref, dp_ref, w_ref, b_ref, o_ref):
    i = pl.program_id(0)
    deg = dp_ref[0, pl.ds(i * BLK, BLK)] + dp_ref[1, pl.ds(i * BLK, BLK)]
    dinv = 1.0 / jnp.maximum(deg, 1.0)
    h = jnp.concatenate([p_ref[0], p_ref[1]], axis=1) * dinv[:, None]
    o_ref[...] = lax.dot_general(
        h, w_ref[...], (((1,), (1,)), ((), ())),
        preferred_element_type=jnp.float32,
    ) + b_ref[...]


def kernel(feature, edge_index, W, b):
    N, D = feature.shape
    H = D // 2
    E = edge_index.shape[1]
    R = ((N + 1 + BLK - 1) // BLK) * BLK

    src = edge_index[0]
    dst = edge_index[1]

    per_w = -(-E // NW)
    C0 = -(-per_w // CHUNK)
    pad0 = NW * C0 * CHUNK - E
    dstp32 = jnp.concatenate(
        [dst, jnp.full((pad0,), N, jnp.int32)]).reshape(NW, C0, CHUNK)

    per_t = -(-E // NS)
    C = -(-per_t // CHUNK)
    pad1 = NS * C * CHUNK - E
    padv = jnp.full((pad1,), N, jnp.int32)
    srcf = jnp.concatenate([src, padv]).reshape(NS, C, CHUNK)
    dstf = jnp.concatenate([dst, padv]).reshape(NS, C, CHUNK)

    ones = jnp.ones((CHUNK,), jnp.float32)
    zrow = jnp.zeros((R,), jnp.float32)
    z2d = jnp.zeros((R // NS, H), jnp.float32)

    dp = _deg_kernel(R, C0)(dstp32, ones, zrow)

    feat_split = pl.pallas_call(
        _prep_body,
        grid=(R // BLK,),
        in_specs=[
            pl.BlockSpec((NC, R), lambda i: (0, 0)),
            pl.BlockSpec((BLK, D), lambda i: (i, 0)),
        ],
        out_specs=pl.BlockSpec((NC, BLK, H), lambda i: (0, i, 0)),
        out_shape=jax.ShapeDtypeStruct((NC, R, H), jnp.float32),
    )(dp, feature)

    partial = _msg_kernel(R, C, H)(srcf, dstf, feat_split, z2d)

    outp = pl.pallas_call(
        _out_body,
        grid=(R // BLK,),
        in_specs=[
            pl.BlockSpec((NC, BLK, H), lambda i: (0, i, 0)),
            pl.BlockSpec((NC, R), lambda i: (0, 0)),
            pl.BlockSpec((D, D), lambda i: (0, 0)),
            pl.BlockSpec((1, D), lambda i: (0, 0)),
        ],
        out_specs=pl.BlockSpec((BLK, D), lambda i: (i, 0)),
        out_shape=jax.ShapeDtypeStruct((N, D), jnp.float32),
    )(partial, dp, W, b.reshape(1, D))

    return outp


# speedup vs baseline: 1.1565x; 1.1055x over previous
import functools

import jax
import jax.numpy as jnp
from jax import lax
from jax.experimental import pallas as pl
from jax.experimental.pallas import tpu as pltpu
from jax.experimental.pallas import tpu_sc as plsc

NC = 2
NS = 16
NW = NC * NS
CHUNK = 128
BLK = 1024


def _mesh():
    return plsc.VectorSubcoreMesh(
        core_axis_name="c", subcore_axis_name="s", num_cores=NC, num_subcores=NS
    )


def _deg_kernel(R, C):
    rows_per = R // NS

    @functools.partial(
        pl.kernel,
        out_type=jax.ShapeDtypeStruct((NC, R), jnp.float32),
        mesh=_mesh(),
        scratch_types=[
            pltpu.VMEM((C, CHUNK), jnp.int32),
            pltpu.VMEM((CHUNK,), jnp.float32),
            pltpu.VMEM((rows_per,), jnp.float32),
            pltpu.VMEM_SHARED((R,), jnp.float32),
        ],
    )
    def deg(dst_hbm, ones_hbm, zrow_hbm, out_hbm, idx_v, ones_v, stage_v, deg_sh):
        c = lax.axis_index("c")
        s = lax.axis_index("s")
        wid = c * NS + s
        base = s * rows_per
        pltpu.sync_copy(zrow_hbm.at[pl.ds(base, rows_per)],
                        deg_sh.at[pl.ds(base, rows_per)])
        pltpu.sync_copy(ones_hbm, ones_v)
        pltpu.sync_copy(dst_hbm.at[wid], idx_v)
        plsc.subcore_barrier()

        def body(g, carry):
            pltpu.sync_copy(ones_v, deg_sh.at[idx_v.at[g]], add=True)
            return carry

        lax.fori_loop(0, C, body, 0)
        plsc.subcore_barrier()
        pltpu.sync_copy(deg_sh.at[pl.ds(base, rows_per)], stage_v)
        pltpu.sync_copy(stage_v, out_hbm.at[c, pl.ds(base, rows_per)])

    return deg


def _msg_kernel(R, C, H):
    rows_per = R // NS

    @functools.partial(
        pl.kernel,
        out_type=jax.ShapeDtypeStruct((NC, R, H), jnp.float32),
        mesh=_mesh(),
        scratch_types=[
            pltpu.VMEM((C, CHUNK), jnp.int32),
            pltpu.VMEM((C, CHUNK), jnp.int32),
            pltpu.VMEM((CHUNK, H), jnp.float32),
            pltpu.SemaphoreType.DMA,
            pltpu.VMEM_SHARED((R, H), jnp.float32),
            pltpu.VMEM_SHARED((R, H), jnp.float32),
        ],
        compiler_params=pltpu.CompilerParams(use_tc_tiling_on_sc=False),
    )
    def msg(src_hbm, dst_hbm, feat_hbm, zrows_hbm, out_hbm,
            sidx_v, didx_v, gbuf, gsem, acc_sh, feat_sh):
        c = lax.axis_index("c")
        s = lax.axis_index("s")
        base = s * rows_per

        pltpu.sync_copy(src_hbm.at[s], sidx_v)
        pltpu.sync_copy(dst_hbm.at[s], didx_v)
        pltpu.sync_copy(feat_hbm.at[c, pl.ds(base, rows_per)],
                        feat_sh.at[pl.ds(base, rows_per)])
        pltpu.sync_copy(zrows_hbm, acc_sh.at[pl.ds(base, rows_per)])
        plsc.subcore_barrier()

        def body(g, carry):
            pltpu.async_copy(feat_sh.at[sidx_v.at[g]],
                             gbuf, gsem).wait()
            pltpu.sync_copy(gbuf, acc_sh.at[didx_v.at[g]], add=True)
            return carry

        lax.fori_loop(0, C, body, 0)
        plsc.subcore_barrier()
        pltpu.sync_copy(acc_sh.at[pl.ds(base, rows_per)],
                        out_hbm.at[c, pl.ds(base, rows_per)])

    return msg


def _prep_body(dp_ref, feat_ref, out_ref):
    i = pl.program_id(0)
    deg = dp_ref[0, pl.ds(i * BLK, BLK)] + dp_ref[1, pl.ds(i * BLK, BLK)]
    scale = lax.rsqrt(deg + 1.0)
    scaled = feat_ref[...] * scale[:, None]
    h = scaled.shape[1] // 2
    out_ref[0] = scaled[:, :h]
    out_ref[1] = scaled[:, h:]


def _out_body(p_ref, dp_ref, w_ref, b_ref, o_ref):
    i = pl.program_id(0)
    deg = dp_ref[0, pl.ds(i * BLK, BLK)] + dp_ref[1, pl.ds(i * BLK, BLK)]
    dinv = 1.0 / jnp.maximum(deg, 1.0)
    h = jnp.concatenate([p_ref[0], p_ref[1]], axis=1) * dinv[:, None]
    o_ref[...] = lax.dot_general(
        h, w_ref[...], (((1,), (1,)), ((), ())),
        preferred_element_type=jnp.float32,
    ) + b_ref[...]


def kernel(feature, edge_index, W, b):
    N, D = feature.shape
    H = D // 2
    E = edge_index.shape[1]
    R = ((N + 1 + BLK - 1) // BLK) * BLK

    src = edge_index[0]
    dst = edge_index[1]

    per_w = -(-E // NW)
    C0 = -(-per_w // CHUNK)
    pad0 = NW * C0 * CHUNK - E
    dstp32 = jnp.concatenate(
        [dst, jnp.full((pad0,), N, jnp.int32)]).reshape(NW, C0, CHUNK)

    per_t = -(-E // NS)
    C = -(-per_t // CHUNK)
    pad1 = NS * C * CHUNK - E
    padv = jnp.full((pad1,), N, jnp.int32)
    srcf = jnp.concatenate([src, padv]).reshape(NS, C, CHUNK)
    dstf = jnp.concatenate([dst, padv]).reshape(NS, C, CHUNK)

    ones = jnp.ones((CHUNK,), jnp.float32)
    zrow = jnp.zeros((R,), jnp.float32)
    z2d = jnp.zeros((R // NS, H), jnp.float32)

    dp = _deg_kernel(R, C0)(dstp32, ones, zrow)

    feat_split = pl.pallas_call(
        _prep_body,
        grid=(R // BLK,),
        in_specs=[
            pl.BlockSpec((NC, R), lambda i: (0, 0)),
            pl.BlockSpec((BLK, D), lambda i: (i, 0)),
        ],
        out_specs=pl.BlockSpec((NC, BLK, H), lambda i: (0, i, 0)),
        out_shape=jax.ShapeDtypeStruct((NC, R, H), jnp.float32),
    )(dp, feature)

    partial = _msg_kernel(R, C, H)(srcf, dstf, feat_split, z2d)

    outp = pl.pallas_call(
        _out_body,
        grid=(R // BLK,),
        in_specs=[
            pl.BlockSpec((NC, BLK, H), lambda i: (0, i, 0)),
            pl.BlockSpec((NC, R), lambda i: (0, 0)),
            pl.BlockSpec((D, D), lambda i: (0, 0)),
            pl.BlockSpec((1, D), lambda i: (0, 0)),
        ],
        out_specs=pl.BlockSpec((BLK, D), lambda i: (i, 0)),
        out_shape=jax.ShapeDtypeStruct((N, D), jnp.float32),
    )(partial, dp, W, b.reshape(1, D))

    return outp


# speedup vs baseline: 1.1825x; 1.0225x over previous
import functools

import jax
import jax.numpy as jnp
from jax import lax
from jax.experimental import pallas as pl
from jax.experimental.pallas import tpu as pltpu
from jax.experimental.pallas import tpu_sc as plsc

NC = 2
NS = 16
NW = NC * NS
CHUNK = 128
BLK = 2560


def _mesh():
    return plsc.VectorSubcoreMesh(
        core_axis_name="c", subcore_axis_name="s", num_cores=NC, num_subcores=NS
    )


def _deg_kernel(R, C):
    rows_per = R // NS

    @functools.partial(
        pl.kernel,
        out_type=jax.ShapeDtypeStruct((NC, R), jnp.float32),
        mesh=_mesh(),
        scratch_types=[
            pltpu.VMEM((C, CHUNK), jnp.int32),
            pltpu.VMEM((CHUNK,), jnp.float32),
            pltpu.VMEM((rows_per,), jnp.float32),
            pltpu.VMEM_SHARED((R,), jnp.float32),
        ],
    )
    def deg(dst_hbm, ones_hbm, zrow_hbm, out_hbm, idx_v, ones_v, stage_v, deg_sh):
        c = lax.axis_index("c")
        s = lax.axis_index("s")
        wid = c * NS + s
        base = s * rows_per
        pltpu.sync_copy(zrow_hbm.at[pl.ds(base, rows_per)],
                        deg_sh.at[pl.ds(base, rows_per)])
        pltpu.sync_copy(ones_hbm, ones_v)
        pltpu.sync_copy(dst_hbm.at[wid], idx_v)
        plsc.subcore_barrier()

        def body(g, carry):
            pltpu.sync_copy(ones_v, deg_sh.at[idx_v.at[g]], add=True)
            return carry

        lax.fori_loop(0, C, body, 0)
        plsc.subcore_barrier()
        pltpu.sync_copy(deg_sh.at[pl.ds(base, rows_per)], stage_v)
        pltpu.sync_copy(stage_v, out_hbm.at[c, pl.ds(base, rows_per)])

    return deg


def _msg_kernel(R, C, H):
    rows_per = R // NS

    @functools.partial(
        pl.kernel,
        out_type=jax.ShapeDtypeStruct((NC, R, H), jnp.float32),
        mesh=_mesh(),
        scratch_types=[
            pltpu.VMEM((C, CHUNK), jnp.int32),
            pltpu.VMEM((C, CHUNK), jnp.int32),
            pltpu.VMEM((CHUNK, H), jnp.float32),
            pltpu.SemaphoreType.DMA,
            pltpu.VMEM_SHARED((R, H), jnp.float32),
            pltpu.VMEM_SHARED((R, H), jnp.float32),
        ],
        compiler_params=pltpu.CompilerParams(use_tc_tiling_on_sc=False),
    )
    def msg(src_hbm, dst_hbm, feat_hbm, zrows_hbm, out_hbm,
            sidx_v, didx_v, gbuf, gsem, acc_sh, feat_sh):
        c = lax.axis_index("c")
        s = lax.axis_index("s")
        base = s * rows_per

        pltpu.sync_copy(src_hbm.at[s], sidx_v)
        pltpu.sync_copy(dst_hbm.at[s], didx_v)
        pltpu.sync_copy(feat_hbm.at[c, pl.ds(base, rows_per)],
                        feat_sh.at[pl.ds(base, rows_per)])
        pltpu.sync_copy(zrows_hbm, acc_sh.at[pl.ds(base, rows_per)])
        plsc.subcore_barrier()

        def body(g, carry):
            pltpu.async_copy(feat_sh.at[sidx_v.at[g]],
                             gbuf, gsem).wait()
            pltpu.sync_copy(gbuf, acc_sh.at[didx_v.at[g]], add=True)
            return carry

        lax.fori_loop(0, C, body, 0)
        plsc.subcore_barrier()
        pltpu.sync_copy(acc_sh.at[pl.ds(base, rows_per)],
                        out_hbm.at[c, pl.ds(base, rows_per)])

    return msg


def _prep_body(dp_ref, feat_ref, out_ref):
    i = pl.program_id(0)
    deg = dp_ref[0, pl.ds(i * BLK, BLK)] + dp_ref[1, pl.ds(i * BLK, BLK)]
    scale = lax.rsqrt(deg + 1.0)
    scaled = feat_ref[...] * scale[:, None]
    h = scaled.shape[1] // 2
    out_ref[0] = scaled[:, :h]
    out_ref[1] = scaled[:, h:]


def _out_body(p_ref, dp_ref, w_ref, b_ref, o_ref):
    i = pl.program_id(0)
    deg = dp_ref[0, pl.ds(i * BLK, BLK)] + dp_ref[1, pl.ds(i * BLK, BLK)]
    dinv = 1.0 / jnp.maximum(deg, 1.0)
    h = jnp.concatenate([p_ref[0], p_ref[1]], axis=1) * dinv[:, None]
    o_ref[...] = lax.dot_general(
        h, w_ref[...], (((1,), (1,)), ((), ())),
        preferred_element_type=jnp.float32,
    ) + b_ref[...]


def kernel(feature, edge_index, W, b):
    N, D = feature.shape
    H = D // 2
    E = edge_index.shape[1]
    R = ((N + 1 + BLK - 1) // BLK) * BLK

    src = edge_index[0]
    dst = edge_index[1]

    per_w = -(-E // NW)
    C0 = -(-per_w // CHUNK)
    pad0 = NW * C0 * CHUNK - E
    dstp32 = jnp.concatenate(
        [dst, jnp.full((pad0,), N, jnp.int32)]).reshape(NW, C0, CHUNK)

    per_t = -(-E // NS)
    C = -(-per_t // CHUNK)
    pad1 = NS * C * CHUNK - E
    padv = jnp.full((pad1,), N, jnp.int32)
    srcf = jnp.concatenate([src, padv]).reshape(NS, C, CHUNK)
    dstf = jnp.concatenate([dst, padv]).reshape(NS, C, CHUNK)

    ones = jnp.ones((CHUNK,), jnp.float32)
    zrow = jnp.zeros((R,), jnp.float32)
    z2d = jnp.zeros((R // NS, H), jnp.float32)

    dp = _deg_kernel(R, C0)(dstp32, ones, zrow)

    feat_split = pl.pallas_call(
        _prep_body,
        grid=(R // BLK,),
        in_specs=[
            pl.BlockSpec((NC, R), lambda i: (0, 0)),
            pl.BlockSpec((BLK, D), lambda i: (i, 0)),
        ],
        out_specs=pl.BlockSpec((NC, BLK, H), lambda i: (0, i, 0)),
        out_shape=jax.ShapeDtypeStruct((NC, R, H), jnp.float32),
    )(dp, feature)

    partial = _msg_kernel(R, C, H)(srcf, dstf, feat_split, z2d)

    outp = pl.pallas_call(
        _out_body,
        grid=(R // BLK,),
        in_specs=[
            pl.BlockSpec((NC, BLK, H), lambda i: (0, i, 0)),
            pl.BlockSpec((NC, R), lambda i: (0, 0)),
            pl.BlockSpec((D, D), lambda i: (0, 0)),
            pl.BlockSpec((1, D), lambda i: (0, 0)),
        ],
        out_specs=pl.BlockSpec((BLK, D), lambda i: (i, 0)),
        out_shape=jax.ShapeDtypeStruct((N, D), jnp.float32),
    )(partial, dp, W, b.reshape(1, D))

    return outp
